# baseline (speedup 1.0000x reference)
import jax
import jax.numpy as jnp
from jax import lax
from jax.experimental import pallas as pl
from jax.experimental.pallas import tpu as pltpu

N_TOK = 20000
N_CLS = 92
K = 64
BLK = 2560
NB = 8
RPB = BLK // 128
NCHUNK = 20
D_H = 256
D_B = 4
B_OFF = 96
BIG = 1 << 30


def _body(lb_blk, h_any, lb_any,
          out_h, out_b, out_l,
          scores, idxs, lbrows, sem_h, sem_lb):
    i = pl.program_id(0)

    @pl.when(i < NB)
    def _phase1():
        x = lb_blk[...]
        sc = jnp.max(x[:, : N_CLS - 1], axis=1)
        tok = i * BLK + lax.broadcasted_iota(jnp.int32, (BLK,), 0)
        sc = jnp.where(tok < N_TOK, sc, -jnp.inf)
        scores[pl.ds(i * RPB, RPB), :] = sc.reshape(RPB, 128)

    @pl.when(i == NB)
    def _phase2():
        lane1 = lax.broadcasted_iota(jnp.int32, (1, 128), 1)
        flat8 = (lax.broadcasted_iota(jnp.int32, (8, 128), 0) * 128
                 + lax.broadcasted_iota(jnp.int32, (8, 128), 1))

        cm = jnp.full((1, 128), -jnp.inf, jnp.float32)
        for c in range(NCHUNK):
            mc = jnp.max(scores[pl.ds(8 * c, 8), :])
            cm = jnp.where(lane1 == c, mc, cm)

        def topk_body(k, cm):
            m = jnp.max(cm)
            c = jnp.min(jnp.where(cm == m, lane1, BIG))
            row = pl.multiple_of(c * 8, 8)
            y = scores[pl.ds(row, 8), :]
            floc = jnp.min(jnp.where(y == m, flat8, BIG))
            tok = c * 1024 + floc
            idxs[k] = tok
            pltpu.make_async_copy(
                h_any.at[pl.ds(tok, 1), :], out_h.at[pl.ds(k, 1), :],
                sem_h).start()
            pltpu.make_async_copy(
                lb_any.at[pl.ds(tok, 1), :], lbrows.at[pl.ds(k, 1), :],
                sem_lb).start()
            y = jnp.where(flat8 == floc, -jnp.inf, y)
            scores[pl.ds(row, 8), :] = y
            return jnp.where(lane1 == c, jnp.max(y), cm)

        lax.fori_loop(0, K, topk_body, cm, unroll=False)

        def gather_wait(k, _):
            tok = idxs[k]
            pltpu.make_async_copy(
                h_any.at[pl.ds(tok, 1), :], out_h.at[pl.ds(k, 1), :],
                sem_h).wait()
            pltpu.make_async_copy(
                lb_any.at[pl.ds(tok, 1), :], lbrows.at[pl.ds(k, 1), :],
                sem_lb).wait()
            return 0

        lax.fori_loop(0, K, gather_wait, 0, unroll=False)

        rows = lbrows[...]
        out_l[...] = rows[:, :N_CLS]
        out_b[...] = rows[:, B_OFF:B_OFF + D_B]


def kernel(h, pred_boxes, pred_logits):
    h2 = h[0]
    b2 = pred_boxes[0]
    l2 = pred_logits[0]

    e_l = jnp.eye(N_CLS, 128, dtype=jnp.float32)
    e_b = jnp.eye(D_B, 128, k=B_OFF, dtype=jnp.float32)
    hp = jax.lax.Precision.HIGHEST
    lb = (jnp.matmul(l2, e_l, precision=hp)
          + jnp.matmul(b2, e_b, precision=hp))

    out_h, out_b, out_l = pl.pallas_call(
        _body,
        grid=(NB + 1,),
        in_specs=[
            pl.BlockSpec((BLK, 128), lambda i: (jnp.minimum(i, NB - 1), 0)),
            pl.BlockSpec(memory_space=pl.ANY),
            pl.BlockSpec(memory_space=pl.ANY),
        ],
        out_specs=[
            pl.BlockSpec((K, D_H), lambda i: (0, 0)),
            pl.BlockSpec((K, D_B), lambda i: (0, 0)),
            pl.BlockSpec((K, N_CLS), lambda i: (0, 0)),
        ],
        out_shape=[
            jax.ShapeDtypeStruct((K, D_H), jnp.float32),
            jax.ShapeDtypeStruct((K, D_B), jnp.float32),
            jax.ShapeDtypeStruct((K, N_CLS), jnp.float32),
        ],
        scratch_shapes=[
            pltpu.VMEM((NCHUNK * 8, 128), jnp.float32),
            pltpu.SMEM((K,), jnp.int32),
            pltpu.VMEM((K, 128), jnp.float32),
            pltpu.SemaphoreType.DMA,
            pltpu.SemaphoreType.DMA,
        ],
        compiler_params=pltpu.CompilerParams(
            dimension_semantics=("arbitrary",),
        ),
    )(lb, h2, lb)

    seq = jnp.concatenate([out_h, out_b, out_l], axis=-1)[None]
    return seq


# speedup vs baseline: 1.3216x; 1.3216x over previous
import jax
import jax.numpy as jnp
from jax import lax
from jax.experimental import pallas as pl
from jax.experimental.pallas import tpu as pltpu

N_TOK = 20000
N_CLS = 92
K = 64
BLK = 5120
NB = 4
D_H = 256
D_B = 4
B_OFF = 96


def _body(lb_blk, h_any, lb_any,
          out_h, out_b, out_l,
          scores, idxs, lbrows, sem_h, sem_lb):
    i = pl.program_id(0)

    @pl.when(i < NB)
    def _phase1():
        x = lb_blk[...]
        sc = jnp.max(x[:, : N_CLS - 1], axis=1)
        tok = i * BLK + lax.broadcasted_iota(jnp.int32, (BLK,), 0)
        sc = jnp.where(tok < N_TOK, sc, -jnp.inf)
        scores[i, :] = sc

    @pl.when(i == NB)
    def _phase2():
        flat = (lax.broadcasted_iota(jnp.int32, (NB, BLK), 0) * BLK
                + lax.broadcasted_iota(jnp.int32, (NB, BLK), 1))

        def topk_body(k, x):
            m = jnp.max(x)
            idx = jnp.min(jnp.where(x == m, flat, jnp.int32(1 << 30)))
            idxs[k] = idx
            pltpu.make_async_copy(
                h_any.at[pl.ds(idx, 1), :], out_h.at[pl.ds(k, 1), :],
                sem_h).start()
            pltpu.make_async_copy(
                lb_any.at[pl.ds(idx, 1), :], lbrows.at[pl.ds(k, 1), :],
                sem_lb).start()
            return jnp.where(flat == idx, -jnp.inf, x)

        lax.fori_loop(0, K, topk_body, scores[...], unroll=False)

        def gather_wait(k, _):
            idx = idxs[k]
            pltpu.make_async_copy(
                h_any.at[pl.ds(idx, 1), :], out_h.at[pl.ds(k, 1), :],
                sem_h).wait()
            pltpu.make_async_copy(
                lb_any.at[pl.ds(idx, 1), :], lbrows.at[pl.ds(k, 1), :],
                sem_lb).wait()
            return 0

        lax.fori_loop(0, K, gather_wait, 0, unroll=False)

        rows = lbrows[...]
        out_l[...] = rows[:, :N_CLS]
        out_b[...] = rows[:, B_OFF:B_OFF + D_B]


def kernel(h, pred_boxes, pred_logits):
    h2 = h[0]
    b2 = pred_boxes[0]
    l2 = pred_logits[0]

    e_l = jnp.eye(N_CLS, 128, dtype=jnp.float32)
    e_b = jnp.eye(D_B, 128, k=B_OFF, dtype=jnp.float32)
    hp = jax.lax.Precision.HIGHEST
    lb = (jnp.matmul(l2, e_l, precision=hp)
          + jnp.matmul(b2, e_b, precision=hp))

    out_h, out_b, out_l = pl.pallas_call(
        _body,
        grid=(NB + 1,),
        in_specs=[
            pl.BlockSpec((BLK, 128), lambda i: (jnp.minimum(i, NB - 1), 0)),
            pl.BlockSpec(memory_space=pl.ANY),
            pl.BlockSpec(memory_space=pl.ANY),
        ],
        out_specs=[
            pl.BlockSpec((K, D_H), lambda i: (0, 0)),
            pl.BlockSpec((K, D_B), lambda i: (0, 0)),
            pl.BlockSpec((K, N_CLS), lambda i: (0, 0)),
        ],
        out_shape=[
            jax.ShapeDtypeStruct((K, D_H), jnp.float32),
            jax.ShapeDtypeStruct((K, D_B), jnp.float32),
            jax.ShapeDtypeStruct((K, N_CLS), jnp.float32),
        ],
        scratch_shapes=[
            pltpu.VMEM((NB, BLK), jnp.float32),
            pltpu.SMEM((K,), jnp.int32),
            pltpu.VMEM((K, 128), jnp.float32),
            pltpu.SemaphoreType.DMA,
            pltpu.SemaphoreType.DMA,
        ],
        compiler_params=pltpu.CompilerParams(
            dimension_semantics=("arbitrary",),
        ),
    )(lb, h2, lb)

    seq = jnp.concatenate([out_h, out_b, out_l], axis=-1)[None]
    return seq


# speedup vs baseline: 1.3762x; 1.0413x over previous
import jax
import jax.numpy as jnp
from jax import lax
from jax.experimental import pallas as pl
from jax.experimental.pallas import tpu as pltpu

N_TOK = 20000
N_CLS = 92
K = 64
BLK = 2560
NB = 8
D_H = 256
D_B = 4
B_OFF = 96


def _body(lb_blk, h_any, lb_any, b_any,
          out_h, out_b, out_l,
          scores, idxs, lbrows, sem_h, sem_lb, sem_b):
    i = pl.program_id(0)

    @pl.when(i < NB)
    def _phase1():
        x = lb_blk[...]
        sc = jnp.max(x[:, : N_CLS - 1], axis=1)
        tok = i * BLK + lax.broadcasted_iota(jnp.int32, (BLK,), 0)
        sc = jnp.where(tok < N_TOK, sc, -jnp.inf)
        scores[i, :] = sc

    @pl.when(i == NB)
    def _phase2():
        flat = (lax.broadcasted_iota(jnp.int32, (NB, BLK), 0) * BLK
                + lax.broadcasted_iota(jnp.int32, (NB, BLK), 1))

        def topk_body(k, x):
            m = jnp.max(x)
            idx = jnp.min(jnp.where(x == m, flat, jnp.int32(1 << 30)))
            idxs[k] = idx
            pltpu.make_async_copy(
                h_any.at[pl.ds(idx, 1), :], out_h.at[pl.ds(k, 1), :],
                sem_h).start()
            pltpu.make_async_copy(
                lb_any.at[pl.ds(idx, 1), :], lbrows.at[pl.ds(k, 1), :],
                sem_lb).start()
            pltpu.make_async_copy(
                b_any.at[pl.ds(idx, 1), :], out_b.at[pl.ds(k, 1), :],
                sem_b).start()
            return jnp.where(flat == idx, -jnp.inf, x)

        lax.fori_loop(0, K, topk_body, scores[...], unroll=False)

        def gather_wait(k, _):
            idx = idxs[k]
            pltpu.make_async_copy(
                h_any.at[pl.ds(idx, 1), :], out_h.at[pl.ds(k, 1), :],
                sem_h).wait()
            pltpu.make_async_copy(
                lb_any.at[pl.ds(idx, 1), :], lbrows.at[pl.ds(k, 1), :],
                sem_lb).wait()
            pltpu.make_async_copy(
                b_any.at[pl.ds(idx, 1), :], out_b.at[pl.ds(k, 1), :],
                sem_b).wait()
            return 0

        lax.fori_loop(0, K, gather_wait, 0, unroll=False)

        out_l[...] = lbrows[:, :N_CLS]


def kernel(h, pred_boxes, pred_logits):
    h2 = h[0]
    b2 = pred_boxes[0]
    l2 = pred_logits[0]

    e_l = jnp.eye(N_CLS, 128, dtype=jnp.float32)
    hp = jax.lax.Precision.HIGHEST
    lb = jnp.matmul(l2, e_l, precision=hp)

    out_h, out_b, out_l = pl.pallas_call(
        _body,
        grid=(NB + 1,),
        in_specs=[
            pl.BlockSpec((BLK, 128), lambda i: (jnp.minimum(i, NB - 1), 0)),
            pl.BlockSpec(memory_space=pl.ANY),
            pl.BlockSpec(memory_space=pl.ANY),
            pl.BlockSpec(memory_space=pl.ANY),
        ],
        out_specs=[
            pl.BlockSpec((K, D_H), lambda i: (0, 0)),
            pl.BlockSpec((K, D_B), lambda i: (0, 0)),
            pl.BlockSpec((K, N_CLS), lambda i: (0, 0)),
        ],
        out_shape=[
            jax.ShapeDtypeStruct((K, D_H), jnp.float32),
            jax.ShapeDtypeStruct((K, D_B), jnp.float32),
            jax.ShapeDtypeStruct((K, N_CLS), jnp.float32),
        ],
        scratch_shapes=[
            pltpu.VMEM((NB, BLK), jnp.float32),
            pltpu.SMEM((K,), jnp.int32),
            pltpu.VMEM((K, 128), jnp.float32),
            pltpu.SemaphoreType.DMA,
            pltpu.SemaphoreType.DMA,
            pltpu.SemaphoreType.DMA,
        ],
        compiler_params=pltpu.CompilerParams(
            dimension_semantics=("arbitrary",),
        ),
    )(lb, h2, lb, b2)

    seq = jnp.concatenate([out_h, out_b, out_l], axis=-1)[None]
    return seq
